# baseline (speedup 1.0000x reference)
import functools

import jax
import jax.numpy as jnp
from jax import lax
from jax.experimental import pallas as pl
from jax.experimental.pallas import tpu as pltpu
from jax.experimental.pallas import tpu_sc as plsc

N = 50000
NPAD = 50048
E = 800000
NW = 32
EPW = E // NW
CH = 125
NCH = EPW // CH
RPT = NPAD // 16
RCH = 136
NRCH = RPT // RCH


NBUF = 8
LOOK = 4


def _seg_pass(table_hbm, out_hbm, zero_hbm, src_v, dst_v, bufs, acc,
              gsems, ssems, zsem, c, s):
    pltpu.async_copy(zero_hbm, acc.at[pl.ds(s * RPT, RPT)], zsem)
    pltpu.make_async_copy(zero_hbm, acc.at[pl.ds(s * RPT, RPT)], zsem).wait()
    plsc.subcore_barrier()

    def _g_start(j, b):
        pltpu.async_copy(table_hbm.at[src_v.at[j]], bufs[b], gsems[b])

    def _g_wait(b):
        pltpu.make_async_copy(table_hbm.at[src_v.at[0]], bufs[b],
                              gsems[b]).wait()

    def _s_start(j, b):
        pltpu.async_copy(bufs[b], acc.at[dst_v.at[j]], ssems[b], add=True)

    def _s_wait(b):
        pltpu.make_async_copy(bufs[b], acc.at[dst_v.at[0]], ssems[b]).wait()

    for j in range(LOOK):
        _g_start(j, j)

    def _body(i, _):
        for off in range(NBUF):
            j = NBUF * i + off
            b = off
            bn = (off + LOOK) % NBUF
            _g_wait(b)
            _s_start(j, b)

            @pl.when(j + LOOK < NCH)
            def _():
                @pl.when(j >= LOOK)
                def _():
                    _s_wait(bn)
                _g_start(j + LOOK, bn)
        return 0
    lax.fori_loop(0, NCH // NBUF, _body, 0)
    for b in range(NBUF):
        _s_wait(b)
    plsc.subcore_barrier()

    r = s * RPT
    pltpu.async_copy(acc.at[pl.ds(r, RPT)], out_hbm.at[c, pl.ds(r, RPT)],
                     zsem)
    pltpu.make_async_copy(acc.at[pl.ds(r, RPT)],
                          out_hbm.at[c, pl.ds(r, RPT)], zsem).wait()


def _make_seg_sum(n_tables):
    W = 16
    mesh = plsc.VectorSubcoreMesh(core_axis_name="c", subcore_axis_name="s")

    @functools.partial(
        pl.kernel,
        out_type=[jax.ShapeDtypeStruct((2, NPAD, W), jnp.float32)
                  for _ in range(n_tables)],
        mesh=mesh,
        compiler_params=pltpu.CompilerParams(use_tc_tiling_on_sc=False),
        scratch_types=[
            pltpu.VMEM((NCH, CH), jnp.int32),
            pltpu.VMEM((NCH, CH), jnp.int32),
        ] + [pltpu.VMEM((CH, W), jnp.float32) for _ in range(NBUF)]
        + [pltpu.VMEM_SHARED((NPAD, W), jnp.float32)]
        + [pltpu.SemaphoreType.DMA for _ in range(2 * NBUF + 1)],
    )
    def seg_sum(*args):
        tables = args[:n_tables]
        src_hbm, dst_hbm, zero_hbm = args[n_tables:n_tables + 3]
        outs = args[n_tables + 3:2 * n_tables + 3]
        rest = args[2 * n_tables + 3:]
        src_v, dst_v = rest[0], rest[1]
        bufs = rest[2:2 + NBUF]
        acc = rest[2 + NBUF]
        gsems = rest[3 + NBUF:3 + 2 * NBUF]
        ssems = rest[3 + 2 * NBUF:3 + 3 * NBUF]
        zsem = rest[3 + 3 * NBUF]
        c = lax.axis_index("c")
        s = lax.axis_index("s")
        w = c * 16 + s
        pltpu.sync_copy(src_hbm.at[w], src_v)
        pltpu.sync_copy(dst_hbm.at[w], dst_v)
        for t in range(n_tables):
            _seg_pass(tables[t], outs[t], zero_hbm, src_v, dst_v, bufs,
                      acc, gsems, ssems, zsem, c, s)

    return seg_sum


_seg_sum_x1 = _make_seg_sum(1)
_seg_sum_x2 = _make_seg_sum(2)

_B = 2000


def _stage_mid_body(pa_ref, pb_ref, x_ref, wl_ref, wr_ref, b96_ref, ab_ref,
                    c16_ref, y1_ref, zrc_ref, rcp_ref):
    agg_a = pa_ref[0] + pa_ref[1]
    agg_b = pb_ref[0] + pb_ref[1]
    cnt = agg_b[:, 8:9]
    rcp = 1.0 / jnp.maximum(cnt, 1.0)
    agg24 = jnp.concatenate([agg_a, agg_b[:, :8]], axis=1)
    h = (jnp.dot(agg24 * rcp, wl_ref[...],
                 preferred_element_type=jnp.float32)
         + jnp.dot(x_ref[...], wr_ref[...],
                   preferred_element_type=jnp.float32)
         + b96_ref[...])
    h = jnp.maximum(h, 0.0)
    ab = jnp.dot(h, ab_ref[...], preferred_element_type=jnp.float32)
    y1_ref[...] = ab[:, :16]
    zrc_ref[...] = ab[:, 16:] + c16_ref[...]
    rcp_ref[...] = rcp


def _stage_mid(parts_a, parts_b, x24, wl, wr, b96, ab, c16):
    return pl.pallas_call(
        _stage_mid_body,
        grid=(N // _B,),
        in_specs=[
            pl.BlockSpec((2, _B, 16), lambda i: (0, i, 0)),
            pl.BlockSpec((2, _B, 16), lambda i: (0, i, 0)),
            pl.BlockSpec((_B, 24), lambda i: (i, 0)),
            pl.BlockSpec((24, 96), lambda i: (0, 0)),
            pl.BlockSpec((24, 96), lambda i: (0, 0)),
            pl.BlockSpec((1, 96), lambda i: (0, 0)),
            pl.BlockSpec((96, 32), lambda i: (0, 0)),
            pl.BlockSpec((1, 16), lambda i: (0, 0)),
        ],
        out_specs=[
            pl.BlockSpec((_B, 16), lambda i: (i, 0)),
            pl.BlockSpec((_B, 16), lambda i: (i, 0)),
            pl.BlockSpec((_B, 1), lambda i: (i, 0)),
        ],
        out_shape=[
            jax.ShapeDtypeStruct((N, 16), jnp.float32),
            jax.ShapeDtypeStruct((N, 16), jnp.float32),
            jax.ShapeDtypeStruct((N, 1), jnp.float32),
        ],
    )(parts_a, parts_b, x24, wl, wr, b96, ab, c16)


def _stage_out_body(parts_ref, zrc_ref, rcp_ref, f2_ref, b2_ref, out_ref):
    agg = parts_ref[0] + parts_ref[1]
    z = jnp.maximum(agg * rcp_ref[...] + zrc_ref[...], 0.0)
    o = jnp.dot(z, f2_ref[...], preferred_element_type=jnp.float32) \
        + b2_ref[...]
    out_ref[0] = o[:, :12]
    out_ref[1] = o[:, 12:]


def _stage_out(parts2, zrc, rcp, f2, b2):
    return pl.pallas_call(
        _stage_out_body,
        grid=(N // _B,),
        in_specs=[
            pl.BlockSpec((2, _B, 16), lambda i: (0, i, 0)),
            pl.BlockSpec((_B, 16), lambda i: (i, 0)),
            pl.BlockSpec((_B, 1), lambda i: (i, 0)),
            pl.BlockSpec((16, 24), lambda i: (0, 0)),
            pl.BlockSpec((1, 24), lambda i: (0, 0)),
        ],
        out_specs=pl.BlockSpec((2, _B, 12), lambda i: (0, i, 0)),
        out_shape=jax.ShapeDtypeStruct((2, N, 12), jnp.float32),
    )(parts2, zrc, rcp, f2, b2)


def kernel(x, edge_index, Wl1, bl1, Wr1, Wl2, bl2, Wr2,
           fc1_W, fc1_b, fc2_W, fc2_b):
    f32 = jnp.float32
    x24 = x.reshape(N, 24)
    xp_a = x24[:, :16]
    xp_b = jnp.concatenate(
        [x24[:, 16:], jnp.ones((N, 1), f32), jnp.zeros((N, 7), f32)], axis=1)
    src3 = edge_index[0].reshape(NW, NCH, CH)
    dst3 = edge_index[1].reshape(NW, NCH, CH)

    eye12 = jnp.eye(12, dtype=f32)
    wl = jnp.einsum('st,gf->sgtf', eye12, Wl1).reshape(24, 96)
    wr = jnp.einsum('st,gf->sgtf', eye12, Wr1).reshape(24, 96)
    b96 = jnp.tile(bl1, 12).reshape(1, 96)
    w1r = fc1_W.reshape(2, 12, 8, 8)
    a_f = jnp.einsum('gf,isfo->sgio', Wl2, w1r).reshape(96, 16)
    b_f = jnp.einsum('gf,isfo->sgio', Wr2, w1r).reshape(96, 16)
    ab = jnp.concatenate([a_f, b_f], axis=1)
    c16 = (jnp.einsum('f,isfo->io', bl2, w1r) + fc1_b).reshape(1, 16)
    f2 = jnp.zeros((16, 24), f32)
    f2 = f2.at[:8, :12].set(fc2_W[0]).at[8:, 12:].set(fc2_W[1])
    b2 = jnp.concatenate([fc2_b[0], fc2_b[1]]).reshape(1, 24)

    zr = jnp.zeros((RPT, 16), f32)
    parts1a, parts1b = _seg_sum_x2(xp_a, xp_b, src3, dst3, zr)
    y1, zrc, rcp = _stage_mid(parts1a, parts1b, x24, wl, wr, b96, ab, c16)
    parts2, = _seg_sum_x1(y1, src3, dst3, zr)
    return _stage_out(parts2, zrc, rcp, f2, b2)


# speedup vs baseline: 1.0639x; 1.0639x over previous
import functools

import jax
import jax.numpy as jnp
from jax import lax
from jax.experimental import pallas as pl
from jax.experimental.pallas import tpu as pltpu
from jax.experimental.pallas import tpu_sc as plsc

N = 50000
NPAD = 50048
E = 800000
NW = 32
EPW = E // NW
CH = 125
NCH = EPW // CH
RPT = NPAD // 16
RCH = 136
NRCH = RPT // RCH


NBUF = 8
LOOK = 4


def _seg_sum_body(table_hbm, src_hbm, dst_hbm, zero_hbm, out_hbm,
                  src_v, dst_v, bufs, acc, gsems, ssems, zsem, W):
    c = lax.axis_index("c")
    s = lax.axis_index("s")
    w = c * 16 + s

    pltpu.async_copy(zero_hbm, acc.at[pl.ds(s * RPT, RPT)], zsem)
    pltpu.sync_copy(src_hbm.at[w], src_v)
    pltpu.sync_copy(dst_hbm.at[w], dst_v)
    pltpu.make_async_copy(zero_hbm, acc.at[pl.ds(s * RPT, RPT)], zsem).wait()
    plsc.subcore_barrier()

    def _g_start(j, b):
        pltpu.async_copy(table_hbm.at[src_v.at[j]], bufs[b], gsems[b])

    def _g_wait(b):
        pltpu.make_async_copy(table_hbm.at[src_v.at[0]], bufs[b],
                              gsems[b]).wait()

    def _s_start(j, b):
        pltpu.async_copy(bufs[b], acc.at[dst_v.at[j]], ssems[b], add=True)

    def _s_wait(b):
        pltpu.make_async_copy(bufs[b], acc.at[dst_v.at[0]], ssems[b]).wait()

    for j in range(LOOK):
        _g_start(j, j)

    def _body(i, _):
        for off in range(NBUF):
            j = NBUF * i + off
            b = off
            bn = (off + LOOK) % NBUF
            _g_wait(b)
            _s_start(j, b)

            @pl.when(j + LOOK < NCH)
            def _():
                @pl.when(j >= LOOK)
                def _():
                    _s_wait(bn)
                _g_start(j + LOOK, bn)
        return 0
    lax.fori_loop(0, NCH // NBUF, _body, 0)
    for b in range(NBUF):
        _s_wait(b)
    plsc.subcore_barrier()

    r = s * RPT
    pltpu.async_copy(acc.at[pl.ds(r, RPT)], out_hbm.at[c, pl.ds(r, RPT)],
                     zsem)
    pltpu.make_async_copy(acc.at[pl.ds(r, RPT)],
                          out_hbm.at[c, pl.ds(r, RPT)], zsem).wait()


def _make_seg_sum(W):
    mesh = plsc.VectorSubcoreMesh(core_axis_name="c", subcore_axis_name="s")

    @functools.partial(
        pl.kernel,
        out_type=jax.ShapeDtypeStruct((2, NPAD, W), jnp.float32),
        mesh=mesh,
        compiler_params=pltpu.CompilerParams(use_tc_tiling_on_sc=False),
        scratch_types=[
            pltpu.VMEM((NCH, CH), jnp.int32),
            pltpu.VMEM((NCH, CH), jnp.int32),
        ] + [pltpu.VMEM((CH, W), jnp.float32) for _ in range(NBUF)]
        + [pltpu.VMEM_SHARED((NPAD, W), jnp.float32)]
        + [pltpu.SemaphoreType.DMA for _ in range(2 * NBUF + 1)],
    )
    def seg_sum(table_hbm, src_hbm, dst_hbm, zero_hbm, out_hbm,
                src_v, dst_v, *rest):
        bufs = rest[:NBUF]
        acc = rest[NBUF]
        gsems = rest[NBUF + 1:2 * NBUF + 1]
        ssems = rest[2 * NBUF + 1:3 * NBUF + 1]
        zsem = rest[3 * NBUF + 1]
        _seg_sum_body(table_hbm, src_hbm, dst_hbm, zero_hbm, out_hbm,
                      src_v, dst_v, bufs, acc, gsems, ssems, zsem, W)

    return seg_sum


_seg_sum_16 = _make_seg_sum(16)

_B = 2000


def _stage_mid_body(pa_ref, pb_ref, x_ref, wl_ref, wr_ref, b96_ref, ab_ref,
                    c16_ref, y1_ref, zrc_ref, rcp_ref):
    agg_a = pa_ref[0] + pa_ref[1]
    agg_b = pb_ref[0] + pb_ref[1]
    cnt = agg_b[:, 8:9]
    rcp = 1.0 / jnp.maximum(cnt, 1.0)
    agg24 = jnp.concatenate([agg_a, agg_b[:, :8]], axis=1)
    h = (jnp.dot(agg24 * rcp, wl_ref[...],
                 preferred_element_type=jnp.float32)
         + jnp.dot(x_ref[...], wr_ref[...],
                   preferred_element_type=jnp.float32)
         + b96_ref[...])
    h = jnp.maximum(h, 0.0)
    ab = jnp.dot(h, ab_ref[...], preferred_element_type=jnp.float32)
    y1_ref[...] = ab[:, :16]
    zrc_ref[...] = ab[:, 16:] + c16_ref[...]
    rcp_ref[...] = rcp


def _stage_mid(parts_a, parts_b, x24, wl, wr, b96, ab, c16):
    return pl.pallas_call(
        _stage_mid_body,
        grid=(N // _B,),
        in_specs=[
            pl.BlockSpec((2, _B, 16), lambda i: (0, i, 0)),
            pl.BlockSpec((2, _B, 16), lambda i: (0, i, 0)),
            pl.BlockSpec((_B, 24), lambda i: (i, 0)),
            pl.BlockSpec((24, 96), lambda i: (0, 0)),
            pl.BlockSpec((24, 96), lambda i: (0, 0)),
            pl.BlockSpec((1, 96), lambda i: (0, 0)),
            pl.BlockSpec((96, 32), lambda i: (0, 0)),
            pl.BlockSpec((1, 16), lambda i: (0, 0)),
        ],
        out_specs=[
            pl.BlockSpec((_B, 16), lambda i: (i, 0)),
            pl.BlockSpec((_B, 16), lambda i: (i, 0)),
            pl.BlockSpec((_B, 1), lambda i: (i, 0)),
        ],
        out_shape=[
            jax.ShapeDtypeStruct((N, 16), jnp.float32),
            jax.ShapeDtypeStruct((N, 16), jnp.float32),
            jax.ShapeDtypeStruct((N, 1), jnp.float32),
        ],
    )(parts_a, parts_b, x24, wl, wr, b96, ab, c16)


def _stage_out_body(parts_ref, zrc_ref, rcp_ref, f2_ref, b2_ref, out_ref):
    agg = parts_ref[0] + parts_ref[1]
    z = jnp.maximum(agg * rcp_ref[...] + zrc_ref[...], 0.0)
    o = jnp.dot(z, f2_ref[...], preferred_element_type=jnp.float32) \
        + b2_ref[...]
    out_ref[0] = o[:, :12]
    out_ref[1] = o[:, 12:]


def _stage_out(parts2, zrc, rcp, f2, b2):
    return pl.pallas_call(
        _stage_out_body,
        grid=(N // _B,),
        in_specs=[
            pl.BlockSpec((2, _B, 16), lambda i: (0, i, 0)),
            pl.BlockSpec((_B, 16), lambda i: (i, 0)),
            pl.BlockSpec((_B, 1), lambda i: (i, 0)),
            pl.BlockSpec((16, 24), lambda i: (0, 0)),
            pl.BlockSpec((1, 24), lambda i: (0, 0)),
        ],
        out_specs=pl.BlockSpec((2, _B, 12), lambda i: (0, i, 0)),
        out_shape=jax.ShapeDtypeStruct((2, N, 12), jnp.float32),
    )(parts2, zrc, rcp, f2, b2)


def kernel(x, edge_index, Wl1, bl1, Wr1, Wl2, bl2, Wr2,
           fc1_W, fc1_b, fc2_W, fc2_b):
    f32 = jnp.float32
    x24 = x.reshape(N, 24)
    xp_a = x24[:, :16]
    xp_b = jnp.concatenate(
        [x24[:, 16:], jnp.ones((N, 1), f32), jnp.zeros((N, 7), f32)], axis=1)
    src3 = edge_index[0].reshape(NW, NCH, CH)
    dst3 = edge_index[1].reshape(NW, NCH, CH)

    eye12 = jnp.eye(12, dtype=f32)
    wl = jnp.einsum('st,gf->sgtf', eye12, Wl1).reshape(24, 96)
    wr = jnp.einsum('st,gf->sgtf', eye12, Wr1).reshape(24, 96)
    b96 = jnp.tile(bl1, 12).reshape(1, 96)
    w1r = fc1_W.reshape(2, 12, 8, 8)
    a_f = jnp.einsum('gf,isfo->sgio', Wl2, w1r).reshape(96, 16)
    b_f = jnp.einsum('gf,isfo->sgio', Wr2, w1r).reshape(96, 16)
    ab = jnp.concatenate([a_f, b_f], axis=1)
    c16 = (jnp.einsum('f,isfo->io', bl2, w1r) + fc1_b).reshape(1, 16)
    f2 = jnp.zeros((16, 24), f32)
    f2 = f2.at[:8, :12].set(fc2_W[0]).at[8:, 12:].set(fc2_W[1])
    b2 = jnp.concatenate([fc2_b[0], fc2_b[1]]).reshape(1, 24)

    zr = jnp.zeros((RPT, 16), f32)
    parts1a = _seg_sum_16(xp_a, src3, dst3, zr)
    parts1b = _seg_sum_16(xp_b, src3, dst3, zr)
    y1, zrc, rcp = _stage_mid(parts1a, parts1b, x24, wl, wr, b96, ab, c16)
    parts2 = _seg_sum_16(y1, src3, dst3, zr)
    return _stage_out(parts2, zrc, rcp, f2, b2)


# speedup vs baseline: 1.0769x; 1.0122x over previous
import functools

import jax
import jax.numpy as jnp
from jax import lax
from jax.experimental import pallas as pl
from jax.experimental.pallas import tpu as pltpu
from jax.experimental.pallas import tpu_sc as plsc

N = 50000
NPAD = 50048
E = 800000
NW = 32
EPW = E // NW
CH = 125
NCH = EPW // CH
RPT = NPAD // 16
RCH = 136
NRCH = RPT // RCH


NBUF = 8
LOOK = 4


def _seg_sum_body(table_hbm, src_hbm, dst_hbm, zero_hbm, out_hbm,
                  src_v, dst_v, bufs, acc, gsems, ssems, zsem, W):
    c = lax.axis_index("c")
    s = lax.axis_index("s")
    w = c * 16 + s

    pltpu.async_copy(zero_hbm, acc.at[pl.ds(s * RPT, RPT)], zsem)
    pltpu.sync_copy(src_hbm.at[w], src_v)
    pltpu.sync_copy(dst_hbm.at[w], dst_v)
    pltpu.make_async_copy(zero_hbm, acc.at[pl.ds(s * RPT, RPT)], zsem).wait()
    plsc.subcore_barrier()

    def _g_start(j, b):
        pltpu.async_copy(table_hbm.at[src_v.at[j]], bufs[b], gsems[b])

    def _g_wait(b):
        pltpu.make_async_copy(table_hbm.at[src_v.at[0]], bufs[b],
                              gsems[b]).wait()

    def _s_start(j, b):
        pltpu.async_copy(bufs[b], acc.at[dst_v.at[j]], ssems[b], add=True)

    def _s_wait(b):
        pltpu.make_async_copy(bufs[b], acc.at[dst_v.at[0]], ssems[b]).wait()

    for j in range(LOOK):
        _g_start(j, j)

    def _body(i, _):
        for off in range(NBUF):
            j = NBUF * i + off
            b = off
            bn = (off + LOOK) % NBUF
            _g_wait(b)
            _s_start(j, b)

            @pl.when(j + LOOK < NCH)
            def _():
                @pl.when(j >= LOOK)
                def _():
                    _s_wait(bn)
                _g_start(j + LOOK, bn)
        return 0
    lax.fori_loop(0, NCH // NBUF, _body, 0)
    for b in range(NBUF):
        _s_wait(b)
    plsc.subcore_barrier()

    r = s * RPT
    pltpu.async_copy(acc.at[pl.ds(r, RPT)], out_hbm.at[c, pl.ds(r, RPT)],
                     zsem)
    pltpu.make_async_copy(acc.at[pl.ds(r, RPT)],
                          out_hbm.at[c, pl.ds(r, RPT)], zsem).wait()


def _make_seg_sum(W):
    mesh = plsc.VectorSubcoreMesh(core_axis_name="c", subcore_axis_name="s")

    @functools.partial(
        pl.kernel,
        out_type=jax.ShapeDtypeStruct((2, NPAD, W), jnp.float32),
        mesh=mesh,
        compiler_params=pltpu.CompilerParams(use_tc_tiling_on_sc=False),
        scratch_types=[
            pltpu.VMEM((NCH, CH), jnp.int32),
            pltpu.VMEM((NCH, CH), jnp.int32),
        ] + [pltpu.VMEM((CH, W), jnp.float32) for _ in range(NBUF)]
        + [pltpu.VMEM_SHARED((NPAD, W), jnp.float32)]
        + [pltpu.SemaphoreType.DMA for _ in range(2 * NBUF + 1)],
    )
    def seg_sum(table_hbm, src_hbm, dst_hbm, zero_hbm, out_hbm,
                src_v, dst_v, *rest):
        bufs = rest[:NBUF]
        acc = rest[NBUF]
        gsems = rest[NBUF + 1:2 * NBUF + 1]
        ssems = rest[2 * NBUF + 1:3 * NBUF + 1]
        zsem = rest[3 * NBUF + 1]
        _seg_sum_body(table_hbm, src_hbm, dst_hbm, zero_hbm, out_hbm,
                      src_v, dst_v, bufs, acc, gsems, ssems, zsem, W)

    return seg_sum


_seg_sum_16 = _make_seg_sum(16)

_B = 5000


def _stage_mid_body(pa_ref, pb_ref, x_ref, wl_ref, wr_ref, b96_ref, ab_ref,
                    c16_ref, y1_ref, zrc_ref, rcp_ref):
    agg_a = pa_ref[0] + pa_ref[1]
    agg_b = pb_ref[0] + pb_ref[1]
    cnt = agg_b[:, 8:9]
    rcp = 1.0 / jnp.maximum(cnt, 1.0)
    agg24 = jnp.concatenate([agg_a, agg_b[:, :8]], axis=1)
    h = (jnp.dot(agg24 * rcp, wl_ref[...],
                 preferred_element_type=jnp.float32)
         + jnp.dot(x_ref[...], wr_ref[...],
                   preferred_element_type=jnp.float32)
         + b96_ref[...])
    h = jnp.maximum(h, 0.0)
    ab = jnp.dot(h, ab_ref[...], preferred_element_type=jnp.float32)
    y1_ref[...] = ab[:, :16]
    zrc_ref[...] = ab[:, 16:] + c16_ref[...]
    rcp_ref[...] = rcp


def _stage_mid(parts_a, parts_b, x24, wl, wr, b96, ab, c16):
    return pl.pallas_call(
        _stage_mid_body,
        grid=(N // _B,),
        in_specs=[
            pl.BlockSpec((2, _B, 16), lambda i: (0, i, 0)),
            pl.BlockSpec((2, _B, 16), lambda i: (0, i, 0)),
            pl.BlockSpec((_B, 24), lambda i: (i, 0)),
            pl.BlockSpec((24, 96), lambda i: (0, 0)),
            pl.BlockSpec((24, 96), lambda i: (0, 0)),
            pl.BlockSpec((1, 96), lambda i: (0, 0)),
            pl.BlockSpec((96, 32), lambda i: (0, 0)),
            pl.BlockSpec((1, 16), lambda i: (0, 0)),
        ],
        out_specs=[
            pl.BlockSpec((_B, 16), lambda i: (i, 0)),
            pl.BlockSpec((_B, 16), lambda i: (i, 0)),
            pl.BlockSpec((_B, 1), lambda i: (i, 0)),
        ],
        out_shape=[
            jax.ShapeDtypeStruct((N, 16), jnp.float32),
            jax.ShapeDtypeStruct((N, 16), jnp.float32),
            jax.ShapeDtypeStruct((N, 1), jnp.float32),
        ],
    )(parts_a, parts_b, x24, wl, wr, b96, ab, c16)


def _stage_out_body(parts_ref, zrc_ref, rcp_ref, f2_ref, b2_ref, out_ref):
    agg = parts_ref[0] + parts_ref[1]
    z = jnp.maximum(agg * rcp_ref[...] + zrc_ref[...], 0.0)
    o = jnp.dot(z, f2_ref[...], preferred_element_type=jnp.float32) \
        + b2_ref[...]
    out_ref[0] = o[:, :12]
    out_ref[1] = o[:, 12:]


def _stage_out(parts2, zrc, rcp, f2, b2):
    return pl.pallas_call(
        _stage_out_body,
        grid=(N // _B,),
        in_specs=[
            pl.BlockSpec((2, _B, 16), lambda i: (0, i, 0)),
            pl.BlockSpec((_B, 16), lambda i: (i, 0)),
            pl.BlockSpec((_B, 1), lambda i: (i, 0)),
            pl.BlockSpec((16, 24), lambda i: (0, 0)),
            pl.BlockSpec((1, 24), lambda i: (0, 0)),
        ],
        out_specs=pl.BlockSpec((2, _B, 12), lambda i: (0, i, 0)),
        out_shape=jax.ShapeDtypeStruct((2, N, 12), jnp.float32),
    )(parts2, zrc, rcp, f2, b2)


def kernel(x, edge_index, Wl1, bl1, Wr1, Wl2, bl2, Wr2,
           fc1_W, fc1_b, fc2_W, fc2_b):
    f32 = jnp.float32
    x24 = x.reshape(N, 24)
    xp_a = x24[:, :16]
    xp_b = jnp.concatenate(
        [x24[:, 16:], jnp.ones((N, 1), f32), jnp.zeros((N, 7), f32)], axis=1)
    src3 = edge_index[0].reshape(NW, NCH, CH)
    dst3 = edge_index[1].reshape(NW, NCH, CH)

    eye12 = jnp.eye(12, dtype=f32)
    wl = jnp.einsum('st,gf->sgtf', eye12, Wl1).reshape(24, 96)
    wr = jnp.einsum('st,gf->sgtf', eye12, Wr1).reshape(24, 96)
    b96 = jnp.tile(bl1, 12).reshape(1, 96)
    w1r = fc1_W.reshape(2, 12, 8, 8)
    a_f = jnp.einsum('gf,isfo->sgio', Wl2, w1r).reshape(96, 16)
    b_f = jnp.einsum('gf,isfo->sgio', Wr2, w1r).reshape(96, 16)
    ab = jnp.concatenate([a_f, b_f], axis=1)
    c16 = (jnp.einsum('f,isfo->io', bl2, w1r) + fc1_b).reshape(1, 16)
    f2 = jnp.zeros((16, 24), f32)
    f2 = f2.at[:8, :12].set(fc2_W[0]).at[8:, 12:].set(fc2_W[1])
    b2 = jnp.concatenate([fc2_b[0], fc2_b[1]]).reshape(1, 24)

    zr = jnp.zeros((RPT, 16), f32)
    parts1a = _seg_sum_16(xp_a, src3, dst3, zr)
    parts1b = _seg_sum_16(xp_b, src3, dst3, zr)
    y1, zrc, rcp = _stage_mid(parts1a, parts1b, x24, wl, wr, b96, ab, c16)
    parts2 = _seg_sum_16(y1, src3, dst3, zr)
    return _stage_out(parts2, zrc, rcp, f2, b2)
